# baseline (speedup 1.0000x reference)
import dataclasses

import jax
import jax.numpy as jnp
import numpy as np
from jax import lax
from jax.experimental import pallas as pl
from jax.experimental.pallas import tpu as pltpu
from jax.experimental.pallas import tpu_sc as plsc

N_NODES = 10000
E_NUM = 320000
F_IN = 128
D_HEAD = 16
N_HEADS = 8
HD = N_HEADS * D_HEAD

NUM_CORES = 2
NUM_SUBCORES = 16
NUM_TILES = NUM_CORES * NUM_SUBCORES
E_PAD = 327680
E_HALF = E_PAD // 2
EW = 128
EPT2 = E_HALF // NUM_TILES
CHUNKS2 = EPT2 // EW
EPC4 = E_PAD // NUM_SUBCORES
EPT4 = E_HALF // NUM_TILES
EW4 = 80
CHUNKS4 = EPT4 // EW4
DCH = 2560
EPT_D = E_PAD // NUM_TILES
N_PAD = 10240
ROWS_PT = N_PAD // NUM_SUBCORES
ZROWS = 32
HPC = N_HEADS // NUM_CORES

_VEC_MESH = plsc.VectorSubcoreMesh(core_axis_name="c", subcore_axis_name="s")

_SC_PARAMS = pltpu.CompilerParams()
if "needs_layout_passes" in pltpu.CompilerParams.__dataclass_fields__:
    _SC_PARAMS = dataclasses.replace(_SC_PARAMS, needs_layout_passes=False)


def _proj_body(x_ref, w_ref, b_ref, o_ref):
    o_ref[...] = (
        jnp.dot(x_ref[...], w_ref[...], preferred_element_type=jnp.float32)
        + b_ref[...]
    )


def _projections(x, w_all, b_all):
    bn = 1000
    return pl.pallas_call(
        _proj_body,
        grid=(N_NODES // bn,),
        in_specs=[
            pl.BlockSpec((bn, F_IN), lambda i: (i, 0)),
            pl.BlockSpec((F_IN, 3 * HD), lambda i: (0, 0)),
            pl.BlockSpec((1, 3 * HD), lambda i: (0, 0)),
        ],
        out_specs=pl.BlockSpec((bn, 3 * HD), lambda i: (i, 0)),
        out_shape=jax.ShapeDtypeStruct((N_NODES, 3 * HD), jnp.float32),
    )(x, w_all, b_all)


def _pmul_body(src_hbm, dst_hbm, k_hbm, q_hbm, p_hbm,
               sall, dall, kb0, kb1, qb0, qb1,
               sk0, sk1, sq0, sq1, sp0, sp1):
    c = lax.axis_index("c")
    s = lax.axis_index("s")
    base = (c * NUM_SUBCORES + s) * EPT2

    kb = [kb0, kb1]
    qb = [qb0, qb1]
    ksem = [sk0, sk1]
    qsem = [sq0, sq1]
    psem = [sp0, sp1]

    pltpu.sync_copy(src_hbm.at[pl.ds(base, EPT2)], sall)
    pltpu.sync_copy(dst_hbm.at[pl.ds(base, EPT2)], dall)

    pltpu.async_copy(k_hbm.at[sall.at[pl.ds(0, EW)]], kb0, sk0)
    pltpu.async_copy(q_hbm.at[dall.at[pl.ds(0, EW)]], qb0, sq0)

    @pl.loop(0, CHUNKS2 // 2)
    def _pair(t):
        for b in range(2):
            j = 2 * t + b
            nb = 1 - b

            @pl.when(j >= 1)
            def _wait_store():
                pltpu.make_async_copy(kb[nb], p_hbm.at[pl.ds(base, EW)],
                                      psem[nb]).wait()

            @pl.when(j < CHUNKS2 - 1)
            def _fire_next():
                nxt = (j + 1) * EW
                pltpu.async_copy(k_hbm.at[sall.at[pl.ds(nxt, EW)]],
                                 kb[nb], ksem[nb])
                pltpu.async_copy(q_hbm.at[dall.at[pl.ds(nxt, EW)]],
                                 qb[nb], qsem[nb])

            pltpu.make_async_copy(k_hbm.at[sall.at[pl.ds(0, EW)]],
                                  kb[b], ksem[b]).wait()
            pltpu.make_async_copy(q_hbm.at[dall.at[pl.ds(0, EW)]],
                                  qb[b], qsem[b]).wait()

            @pl.loop(0, EW)
            def _row(r):
                for col in range(0, HD, 16):
                    kb[b][r, pl.ds(col, 16)] = (
                        kb[b][r, pl.ds(col, 16)] * qb[b][r, pl.ds(col, 16)]
                    )

            pltpu.async_copy(kb[b], p_hbm.at[pl.ds(base + j * EW, EW)],
                             psem[b])

    pltpu.make_async_copy(kb1, p_hbm.at[pl.ds(base, EW)], sp1).wait()


def _pmul(src, dst, k_t, q_t):
    kern = pl.kernel(
        _pmul_body,
        out_type=jax.ShapeDtypeStruct((E_HALF, HD), jnp.float32),
        mesh=_VEC_MESH,
        scratch_types=[
            pltpu.VMEM((EPT2,), jnp.int32),
            pltpu.VMEM((EPT2,), jnp.int32),
            pltpu.VMEM((EW, HD), jnp.float32),
            pltpu.VMEM((EW, HD), jnp.float32),
            pltpu.VMEM((EW, HD), jnp.float32),
            pltpu.VMEM((EW, HD), jnp.float32),
            pltpu.SemaphoreType.DMA,
            pltpu.SemaphoreType.DMA,
            pltpu.SemaphoreType.DMA,
            pltpu.SemaphoreType.DMA,
            pltpu.SemaphoreType.DMA,
            pltpu.SemaphoreType.DMA,
        ],
    )
    return kern(src, dst, k_t, q_t)


def _escore_body(ea_ref, p_ref, we_ref, be_ref, s_ref, o_ref):
    eh = (
        jnp.dot(ea_ref[...], we_ref[...], preferred_element_type=jnp.float32)
        + be_ref[...]
    )
    t = eh * p_ref[...]
    s = jnp.dot(t, s_ref[...], preferred_element_type=jnp.float32)
    o_ref[...] = jnp.exp(s)


def _escore(edge_attr, p, we, be, s16, row0, nrows):
    be_blk = 1280
    off = row0 // be_blk
    return pl.pallas_call(
        _escore_body,
        grid=(nrows // be_blk,),
        in_specs=[
            pl.BlockSpec((be_blk, F_IN), lambda i: (i + off, 0)),
            pl.BlockSpec((be_blk, HD), lambda i: (i, 0)),
            pl.BlockSpec((F_IN, HD), lambda i: (0, 0)),
            pl.BlockSpec((1, HD), lambda i: (0, 0)),
            pl.BlockSpec((HD, 16), lambda i: (0, 0)),
        ],
        out_specs=pl.BlockSpec((be_blk, 16), lambda i: (i, 0)),
        out_shape=jax.ShapeDtypeStruct((nrows, 16), jnp.float32),
    )(edge_attr, p, we, be, s16)


def _transpose_body(x_ref, o_ref):
    o_ref[...] = x_ref[...].T


def _transpose_ex(ex_pad2d):
    bt = 2048
    return pl.pallas_call(
        _transpose_body,
        grid=(E_PAD // bt,),
        in_specs=[pl.BlockSpec((bt, 16), lambda i: (i, 0))],
        out_specs=pl.BlockSpec((16, bt), lambda i: (0, i)),
        out_shape=jax.ShapeDtypeStruct((16, E_PAD), jnp.float32),
    )(ex_pad2d)


def _scatter_body(src_hbm, dst_hbm, ex_hbm, v_hbm, out_hbm,
                  sall, didx0, didx1, exb0, exb1, vb0, vb1, zwv, wv_acc,
                  si0, si1, se0, se1, sv0, sv1, ss0, ss1):
    c = lax.axis_index("c")
    s = lax.axis_index("s")
    base = (c * NUM_SUBCORES + s) * EPT4
    row0 = s * ROWS_PT

    didx = [didx0, didx1]
    exb = [exb0, exb1]
    vb = [vb0, vb1]
    isem = [si0, si1]
    esem = [se0, se1]
    vsem = [sv0, sv1]
    ssem = [ss0, ss1]

    @pl.loop(0, ZROWS)
    def _zrow(r):
        for col in range(0, HD, 16):
            zwv[r, pl.ds(col, 16)] = jnp.zeros((16,), jnp.float32)

    for i in range(ROWS_PT // ZROWS):
        pltpu.sync_copy(zwv, wv_acc.at[pl.ds(row0 + i * ZROWS, ZROWS)])

    pltpu.sync_copy(src_hbm.at[pl.ds(base, EPT4)], sall)

    plsc.subcore_barrier()

    pltpu.async_copy(dst_hbm.at[pl.ds(base, EW4)], didx0, si0)
    pltpu.async_copy(ex_hbm.at[pl.ds(base * 16, EW4 * 16)], exb0, se0)
    pltpu.async_copy(v_hbm.at[sall.at[pl.ds(0, EW4)]], vb0, sv0)

    @pl.loop(0, CHUNKS4 // 2)
    def _pair(t):
        for b in range(2):
            j = 2 * t + b
            nb = 1 - b

            @pl.when(j >= 1)
            def _wait_scatter():
                pltpu.make_async_copy(vb[nb], wv_acc.at[didx[nb]],
                                      ssem[nb]).wait()

            @pl.when(j < CHUNKS4 - 1)
            def _fire_next():
                nxt = (j + 1) * EW4
                pltpu.async_copy(dst_hbm.at[pl.ds(base + nxt, EW4)],
                                 didx[nb], isem[nb])
                pltpu.async_copy(ex_hbm.at[pl.ds((base + nxt) * 16, EW4 * 16)],
                                 exb[nb], esem[nb])
                pltpu.async_copy(v_hbm.at[sall.at[pl.ds(nxt, EW4)]],
                                 vb[nb], vsem[nb])

            pltpu.make_async_copy(dst_hbm.at[pl.ds(base, EW4)],
                                  didx[b], isem[b]).wait()
            pltpu.make_async_copy(ex_hbm.at[pl.ds(base * 16, EW4 * 16)],
                                  exb[b], esem[b]).wait()
            pltpu.make_async_copy(v_hbm.at[sall.at[pl.ds(0, EW4)]],
                                  vb[b], vsem[b]).wait()

            @pl.loop(0, EW4)
            def _row(r):
                erow = exb[b][pl.ds(r * 16, 16)]
                for hh in range(N_HEADS):
                    exv = jnp.full((16,), erow[hh], dtype=jnp.float32)
                    vb[b][r, pl.ds(hh * D_HEAD, 16)] = (
                        vb[b][r, pl.ds(hh * D_HEAD, 16)] * exv
                    )

            pltpu.async_copy(vb[b], wv_acc.at[didx[b]], ssem[b], add=True)

    pltpu.make_async_copy(vb1, wv_acc.at[didx1], ss1).wait()

    plsc.subcore_barrier()
    pltpu.sync_copy(wv_acc.at[pl.ds(row0, ROWS_PT)],
                    out_hbm.at[c, pl.ds(row0, ROWS_PT)])


def _scatter(src, dst, ex, v_t):
    kern = pl.kernel(
        _scatter_body,
        out_type=jax.ShapeDtypeStruct((NUM_CORES, N_PAD, HD), jnp.float32),
        mesh=_VEC_MESH,
        scratch_types=[
            pltpu.VMEM((EPT4,), jnp.int32),
            pltpu.VMEM((EW4,), jnp.int32),
            pltpu.VMEM((EW4,), jnp.int32),
            pltpu.VMEM((EW4 * 16,), jnp.float32),
            pltpu.VMEM((EW4 * 16,), jnp.float32),
            pltpu.VMEM((EW4, HD), jnp.float32),
            pltpu.VMEM((EW4, HD), jnp.float32),
            pltpu.VMEM((ZROWS, HD), jnp.float32),
            pltpu.VMEM_SHARED((N_PAD, HD), jnp.float32),
            pltpu.SemaphoreType.DMA,
            pltpu.SemaphoreType.DMA,
            pltpu.SemaphoreType.DMA,
            pltpu.SemaphoreType.DMA,
            pltpu.SemaphoreType.DMA,
            pltpu.SemaphoreType.DMA,
            pltpu.SemaphoreType.DMA,
            pltpu.SemaphoreType.DMA,
        ],
    )
    return kern(src, dst, ex, v_t)


def _den_body(dst_hbm, ext_hbm, out_hbm, den, didx, eh0, eh1, sa, sb):
    c = lax.axis_index("c")
    s = lax.axis_index("s")
    wid = c * NUM_SUBCORES + s
    base = wid * EPT_D
    ehb = [eh0, eh1]
    sem = [sa, sb]
    nh = EPT_D // DCH * N_HEADS

    @pl.loop(0, N_NODES * N_HEADS // 16)
    def _z(i):
        den[pl.ds(i * 16, 16)] = jnp.zeros((16,), jnp.float32)

    pltpu.sync_copy(dst_hbm.at[pl.ds(base, EPT_D)], didx)
    pltpu.async_copy(ext_hbm.at[pl.ds(base, DCH)], eh0, sa)

    @pl.loop(0, nh // 2)
    def _pair(t):
        for b in range(2):
            i = 2 * t + b
            nb = 1 - b

            @pl.when(i < nh - 1)
            def _fire():
                i1 = i + 1
                j1 = i1 // N_HEADS
                h1 = i1 % N_HEADS
                pltpu.async_copy(
                    ext_hbm.at[pl.ds(h1 * E_PAD + base + j1 * DCH, DCH)],
                    ehb[nb], sem[nb])

            pltpu.make_async_copy(ext_hbm.at[pl.ds(base, DCH)],
                                  ehb[b], sem[b]).wait()
            j = i // N_HEADS
            h = i % N_HEADS
            doff = j * DCH

            @pl.loop(0, DCH // 16)
            def _vec(v):
                dvec = didx[pl.ds(doff + v * 16, 16)]
                idx = dvec * N_HEADS + h
                ev = ehb[b][pl.ds(v * 16, 16)]
                plsc.addupdate_scatter(den, [idx], ev)

    pltpu.sync_copy(den, out_hbm.at[wid])


def _den(dst, ext):
    kern = pl.kernel(
        _den_body,
        out_type=jax.ShapeDtypeStruct((NUM_TILES, N_NODES * N_HEADS),
                                      jnp.float32),
        mesh=_VEC_MESH,
        compiler_params=_SC_PARAMS,
        scratch_types=[
            pltpu.VMEM((N_NODES * N_HEADS,), jnp.float32),
            pltpu.VMEM((EPT_D,), jnp.int32),
            pltpu.VMEM((DCH,), jnp.float32),
            pltpu.VMEM((DCH,), jnp.float32),
            pltpu.SemaphoreType.DMA,
            pltpu.SemaphoreType.DMA,
        ],
    )
    return kern(dst, ext)


def _combine_body(p_ref, d_ref, t_ref, o_ref):
    eps = 1e-16
    wv = (p_ref[0] + p_ref[1]) + (p_ref[2] + p_ref[3])
    den = jnp.sum(d_ref[...], axis=0)
    rep = jnp.dot(1.0 / (den + eps), t_ref[...],
                  preferred_element_type=jnp.float32)
    o_ref[...] = wv * rep


def _combine(pm, denp, t_ind):
    bn = 1000
    return pl.pallas_call(
        _combine_body,
        grid=(N_NODES // bn,),
        in_specs=[
            pl.BlockSpec((2 * NUM_CORES, bn, HD), lambda i: (0, i, 0)),
            pl.BlockSpec((NUM_TILES, bn, N_HEADS), lambda i: (0, i, 0)),
            pl.BlockSpec((N_HEADS, HD), lambda i: (0, 0)),
        ],
        out_specs=pl.BlockSpec((bn, HD), lambda i: (i, 0)),
        out_shape=jax.ShapeDtypeStruct((N_NODES, HD), jnp.float32),
    )(pm, denp, t_ind)


def kernel(x, edge_attr, edge_index, Wq, bq, Wk, bk, We, be, Wv, bv):
    scale = 1.0 / np.sqrt(D_HEAD)
    w_all = jnp.concatenate([Wq * scale, Wk, Wv], axis=1)
    b_all = jnp.concatenate([bq * scale, bk, bv]).reshape(1, 3 * HD)

    proj = _projections(x, w_all, b_all)
    q_t = proj[:, :HD]
    k_t = proj[:, HD:2 * HD]
    v_t = proj[:, 2 * HD:]

    src = edge_index[0]
    dst = edge_index[1]
    n_extra = E_PAD - E_NUM
    zpad = jnp.zeros((n_extra,), dtype=src.dtype)
    src_pad = jnp.concatenate([src, zpad])
    dstg_pad = jnp.concatenate([dst, zpad])
    dsts_pad = jnp.concatenate([dst, zpad + N_NODES])

    s16 = np.zeros((HD, 16), dtype=np.float32)
    for h in range(N_HEADS):
        s16[h * D_HEAD:(h + 1) * D_HEAD, h] = 1.0
    s16 = jnp.asarray(s16)
    be2 = be.reshape(1, HD)

    p_a = _pmul(src_pad[:E_HALF], dstg_pad[:E_HALF], k_t, q_t)
    p_b = _pmul(src_pad[E_HALF:], dstg_pad[E_HALF:], k_t, q_t)
    ex_a = _escore(edge_attr, p_a, We, be2, s16, 0, E_HALF)
    nreal_b = E_NUM - E_HALF
    ex_b = _escore(edge_attr, p_b, We, be2, s16, E_HALF, nreal_b)
    ex_b_pad = jnp.concatenate([ex_b, jnp.zeros((n_extra, 16), jnp.float32)])

    pm_a = _scatter(src_pad[:E_HALF], dsts_pad[:E_HALF],
                    ex_a.reshape(E_HALF * 16), v_t)
    pm_b = _scatter(src_pad[E_HALF:], dsts_pad[E_HALF:],
                    ex_b_pad.reshape(E_HALF * 16), v_t)
    pm = jnp.concatenate([pm_a, pm_b])

    ex_pad2d = jnp.concatenate([ex_a, ex_b_pad])
    ext_pad = _transpose_ex(ex_pad2d).reshape(16 * E_PAD)
    denp = _den(dstg_pad, ext_pad)

    t_ind = np.zeros((N_HEADS, HD), dtype=np.float32)
    for h in range(N_HEADS):
        t_ind[h, h * D_HEAD:(h + 1) * D_HEAD] = 1.0
    wv = _combine(pm, denp.reshape(NUM_TILES, N_NODES, N_HEADS),
                  jnp.asarray(t_ind))

    return wv.reshape(N_NODES, N_HEADS, D_HEAD)


# speedup vs baseline: 1.2267x; 1.2267x over previous
import dataclasses

import jax
import jax.numpy as jnp
import numpy as np
from jax import lax
from jax.experimental import pallas as pl
from jax.experimental.pallas import tpu as pltpu
from jax.experimental.pallas import tpu_sc as plsc

N_NODES = 10000
E_NUM = 320000
F_IN = 128
D_HEAD = 16
N_HEADS = 8
HD = N_HEADS * D_HEAD

NUM_CORES = 2
NUM_SUBCORES = 16
NUM_TILES = NUM_CORES * NUM_SUBCORES
E_PAD = 327680
EW = 128
EPT2 = E_PAD // NUM_TILES
CHUNKS2 = EPT2 // EW
EPC4 = E_PAD // NUM_SUBCORES
EPT4 = E_PAD // NUM_TILES
EW4 = 80
CHUNKS4 = EPT4 // EW4
DCH = 2560
EPT_D = E_PAD // NUM_TILES
N_PAD = 10240
ROWS_PT = N_PAD // NUM_SUBCORES
ZROWS = 32
HPC = N_HEADS // NUM_CORES

_VEC_MESH = plsc.VectorSubcoreMesh(core_axis_name="c", subcore_axis_name="s")

_SC_PARAMS = pltpu.CompilerParams()
if "needs_layout_passes" in pltpu.CompilerParams.__dataclass_fields__:
    _SC_PARAMS = dataclasses.replace(_SC_PARAMS, needs_layout_passes=False)


def _proj_body(x_ref, w_ref, b_ref, o_ref):
    o_ref[...] = (
        jnp.dot(x_ref[...], w_ref[...], preferred_element_type=jnp.float32)
        + b_ref[...]
    )


def _projections(x, w_all, b_all):
    bn = 1000
    return pl.pallas_call(
        _proj_body,
        grid=(N_NODES // bn,),
        in_specs=[
            pl.BlockSpec((bn, F_IN), lambda i: (i, 0)),
            pl.BlockSpec((F_IN, 3 * HD), lambda i: (0, 0)),
            pl.BlockSpec((1, 3 * HD), lambda i: (0, 0)),
        ],
        out_specs=pl.BlockSpec((bn, 3 * HD), lambda i: (i, 0)),
        out_shape=jax.ShapeDtypeStruct((N_NODES, 3 * HD), jnp.float32),
    )(x, w_all, b_all)


def _pmul_body(src_hbm, dst_hbm, k_hbm, q_hbm, p_hbm,
               sall, dall, kb0, kb1, qb0, qb1,
               sk0, sk1, sq0, sq1, sp0, sp1):
    c = lax.axis_index("c")
    s = lax.axis_index("s")
    base = (c * NUM_SUBCORES + s) * EPT2

    kb = [kb0, kb1]
    qb = [qb0, qb1]
    ksem = [sk0, sk1]
    qsem = [sq0, sq1]
    psem = [sp0, sp1]

    pltpu.sync_copy(src_hbm.at[pl.ds(base, EPT2)], sall)
    pltpu.sync_copy(dst_hbm.at[pl.ds(base, EPT2)], dall)

    pltpu.async_copy(k_hbm.at[sall.at[pl.ds(0, EW)]], kb0, sk0)
    pltpu.async_copy(q_hbm.at[dall.at[pl.ds(0, EW)]], qb0, sq0)

    @pl.loop(0, CHUNKS2 // 2)
    def _pair(t):
        for b in range(2):
            j = 2 * t + b
            nb = 1 - b

            @pl.when(j >= 1)
            def _wait_store():
                pltpu.make_async_copy(kb[nb], p_hbm.at[pl.ds(base, EW)],
                                      psem[nb]).wait()

            @pl.when(j < CHUNKS2 - 1)
            def _fire_next():
                nxt = (j + 1) * EW
                pltpu.async_copy(k_hbm.at[sall.at[pl.ds(nxt, EW)]],
                                 kb[nb], ksem[nb])
                pltpu.async_copy(q_hbm.at[dall.at[pl.ds(nxt, EW)]],
                                 qb[nb], qsem[nb])

            pltpu.make_async_copy(k_hbm.at[sall.at[pl.ds(0, EW)]],
                                  kb[b], ksem[b]).wait()
            pltpu.make_async_copy(q_hbm.at[dall.at[pl.ds(0, EW)]],
                                  qb[b], qsem[b]).wait()

            @pl.loop(0, EW)
            def _row(r):
                for col in range(0, HD, 16):
                    kb[b][r, pl.ds(col, 16)] = (
                        kb[b][r, pl.ds(col, 16)] * qb[b][r, pl.ds(col, 16)]
                    )

            pltpu.async_copy(kb[b], p_hbm.at[pl.ds(base + j * EW, EW)],
                             psem[b])

    pltpu.make_async_copy(kb1, p_hbm.at[pl.ds(base, EW)], sp1).wait()


def _pmul(src, dst, k_t, q_t):
    kern = pl.kernel(
        _pmul_body,
        out_type=jax.ShapeDtypeStruct((E_PAD, HD), jnp.float32),
        mesh=_VEC_MESH,
        scratch_types=[
            pltpu.VMEM((EPT2,), jnp.int32),
            pltpu.VMEM((EPT2,), jnp.int32),
            pltpu.VMEM((EW, HD), jnp.float32),
            pltpu.VMEM((EW, HD), jnp.float32),
            pltpu.VMEM((EW, HD), jnp.float32),
            pltpu.VMEM((EW, HD), jnp.float32),
            pltpu.SemaphoreType.DMA,
            pltpu.SemaphoreType.DMA,
            pltpu.SemaphoreType.DMA,
            pltpu.SemaphoreType.DMA,
            pltpu.SemaphoreType.DMA,
            pltpu.SemaphoreType.DMA,
        ],
    )
    return kern(src, dst, k_t, q_t)


def _escore_body(ea_ref, p_ref, we_ref, be_ref, s_ref, o_ref):
    eh = (
        jnp.dot(ea_ref[...], we_ref[...], preferred_element_type=jnp.float32)
        + be_ref[...]
    )
    t = eh * p_ref[...]
    s = jnp.dot(t, s_ref[...], preferred_element_type=jnp.float32)
    o_ref[...] = jnp.exp(s)


def _escore(edge_attr, p, we, be, s16):
    be_blk = 2000
    return pl.pallas_call(
        _escore_body,
        grid=(E_NUM // be_blk,),
        in_specs=[
            pl.BlockSpec((be_blk, F_IN), lambda i: (i, 0)),
            pl.BlockSpec((be_blk, HD), lambda i: (i, 0)),
            pl.BlockSpec((F_IN, HD), lambda i: (0, 0)),
            pl.BlockSpec((1, HD), lambda i: (0, 0)),
            pl.BlockSpec((HD, 16), lambda i: (0, 0)),
        ],
        out_specs=pl.BlockSpec((be_blk, 16), lambda i: (i, 0)),
        out_shape=jax.ShapeDtypeStruct((E_NUM, 16), jnp.float32),
    )(edge_attr, p, we, be, s16)


def _transpose_body(x_ref, o_ref):
    o_ref[...] = x_ref[...].T


def _transpose_ex(ex_pad2d):
    bt = 2048
    return pl.pallas_call(
        _transpose_body,
        grid=(E_PAD // bt,),
        in_specs=[pl.BlockSpec((bt, 16), lambda i: (i, 0))],
        out_specs=pl.BlockSpec((16, bt), lambda i: (0, i)),
        out_shape=jax.ShapeDtypeStruct((16, E_PAD), jnp.float32),
    )(ex_pad2d)


def _scatter_body(src_hbm, dst_hbm, ex_hbm, v_hbm, out_hbm,
                  sall, didx0, didx1, exb0, exb1, vb0, vb1, zwv, wv_acc,
                  si0, si1, se0, se1, sv0, sv1, ss0, ss1):
    c = lax.axis_index("c")
    s = lax.axis_index("s")
    base = (c * NUM_SUBCORES + s) * EPT4
    row0 = s * ROWS_PT

    didx = [didx0, didx1]
    exb = [exb0, exb1]
    vb = [vb0, vb1]
    isem = [si0, si1]
    esem = [se0, se1]
    vsem = [sv0, sv1]
    ssem = [ss0, ss1]

    @pl.loop(0, ZROWS)
    def _zrow(r):
        for col in range(0, HD, 16):
            zwv[r, pl.ds(col, 16)] = jnp.zeros((16,), jnp.float32)

    for i in range(ROWS_PT // ZROWS):
        pltpu.sync_copy(zwv, wv_acc.at[pl.ds(row0 + i * ZROWS, ZROWS)])

    pltpu.sync_copy(src_hbm.at[pl.ds(base, EPC4)], sall)

    plsc.subcore_barrier()

    pltpu.async_copy(dst_hbm.at[pl.ds(base, EW4)], didx0, si0)
    pltpu.async_copy(ex_hbm.at[pl.ds(base * 16, EW4 * 16)], exb0, se0)
    pltpu.async_copy(v_hbm.at[sall.at[pl.ds(0, EW4)]], vb0, sv0)

    @pl.loop(0, CHUNKS4 // 2)
    def _pair(t):
        for b in range(2):
            j = 2 * t + b
            nb = 1 - b

            @pl.when(j >= 1)
            def _wait_scatter():
                pltpu.make_async_copy(vb[nb], wv_acc.at[didx[nb]],
                                      ssem[nb]).wait()

            @pl.when(j < CHUNKS4 - 1)
            def _fire_next():
                nxt = (j + 1) * EW4
                pltpu.async_copy(dst_hbm.at[pl.ds(base + nxt, EW4)],
                                 didx[nb], isem[nb])
                pltpu.async_copy(ex_hbm.at[pl.ds((base + nxt) * 16, EW4 * 16)],
                                 exb[nb], esem[nb])
                pltpu.async_copy(v_hbm.at[sall.at[pl.ds(nxt, EW4)]],
                                 vb[nb], vsem[nb])

            pltpu.make_async_copy(dst_hbm.at[pl.ds(base, EW4)],
                                  didx[b], isem[b]).wait()
            pltpu.make_async_copy(ex_hbm.at[pl.ds(base * 16, EW4 * 16)],
                                  exb[b], esem[b]).wait()
            pltpu.make_async_copy(v_hbm.at[sall.at[pl.ds(0, EW4)]],
                                  vb[b], vsem[b]).wait()

            @pl.loop(0, EW4)
            def _row(r):
                erow = exb[b][pl.ds(r * 16, 16)]
                for hh in range(N_HEADS):
                    exv = jnp.full((16,), erow[hh], dtype=jnp.float32)
                    vb[b][r, pl.ds(hh * D_HEAD, 16)] = (
                        vb[b][r, pl.ds(hh * D_HEAD, 16)] * exv
                    )

            pltpu.async_copy(vb[b], wv_acc.at[didx[b]], ssem[b], add=True)

    pltpu.make_async_copy(vb1, wv_acc.at[didx1], ss1).wait()

    plsc.subcore_barrier()
    pltpu.sync_copy(wv_acc.at[pl.ds(row0, ROWS_PT)],
                    out_hbm.at[c, pl.ds(row0, ROWS_PT)])


def _scatter(src, dst, ex, v_t):
    kern = pl.kernel(
        _scatter_body,
        out_type=jax.ShapeDtypeStruct((NUM_CORES, N_PAD, HD), jnp.float32),
        mesh=_VEC_MESH,
        scratch_types=[
            pltpu.VMEM((EPC4,), jnp.int32),
            pltpu.VMEM((EW4,), jnp.int32),
            pltpu.VMEM((EW4,), jnp.int32),
            pltpu.VMEM((EW4 * 16,), jnp.float32),
            pltpu.VMEM((EW4 * 16,), jnp.float32),
            pltpu.VMEM((EW4, HD), jnp.float32),
            pltpu.VMEM((EW4, HD), jnp.float32),
            pltpu.VMEM((ZROWS, HD), jnp.float32),
            pltpu.VMEM_SHARED((N_PAD, HD), jnp.float32),
            pltpu.SemaphoreType.DMA,
            pltpu.SemaphoreType.DMA,
            pltpu.SemaphoreType.DMA,
            pltpu.SemaphoreType.DMA,
            pltpu.SemaphoreType.DMA,
            pltpu.SemaphoreType.DMA,
            pltpu.SemaphoreType.DMA,
            pltpu.SemaphoreType.DMA,
        ],
    )
    return kern(src, dst, ex, v_t)


def _den_body(dst_hbm, ext_hbm, out_hbm, den, didx, eh0, eh1, sa, sb):
    c = lax.axis_index("c")
    s = lax.axis_index("s")
    wid = c * NUM_SUBCORES + s
    base = wid * EPT_D
    ehb = [eh0, eh1]
    sem = [sa, sb]
    nh = EPT_D // DCH * N_HEADS

    @pl.loop(0, N_NODES * N_HEADS // 16)
    def _z(i):
        den[pl.ds(i * 16, 16)] = jnp.zeros((16,), jnp.float32)

    pltpu.sync_copy(dst_hbm.at[pl.ds(base, EPT_D)], didx)
    pltpu.async_copy(ext_hbm.at[pl.ds(base, DCH)], eh0, sa)

    @pl.loop(0, nh // 2)
    def _pair(t):
        for b in range(2):
            i = 2 * t + b
            nb = 1 - b

            @pl.when(i < nh - 1)
            def _fire():
                i1 = i + 1
                j1 = i1 // N_HEADS
                h1 = i1 % N_HEADS
                pltpu.async_copy(
                    ext_hbm.at[pl.ds(h1 * E_PAD + base + j1 * DCH, DCH)],
                    ehb[nb], sem[nb])

            pltpu.make_async_copy(ext_hbm.at[pl.ds(base, DCH)],
                                  ehb[b], sem[b]).wait()
            j = i // N_HEADS
            h = i % N_HEADS
            doff = j * DCH

            @pl.loop(0, DCH // 16)
            def _vec(v):
                dvec = didx[pl.ds(doff + v * 16, 16)]
                idx = dvec * N_HEADS + h
                ev = ehb[b][pl.ds(v * 16, 16)]
                plsc.addupdate_scatter(den, [idx], ev)

    pltpu.sync_copy(den, out_hbm.at[wid])


def _den(dst, ext):
    kern = pl.kernel(
        _den_body,
        out_type=jax.ShapeDtypeStruct((NUM_TILES, N_NODES * N_HEADS),
                                      jnp.float32),
        mesh=_VEC_MESH,
        compiler_params=_SC_PARAMS,
        scratch_types=[
            pltpu.VMEM((N_NODES * N_HEADS,), jnp.float32),
            pltpu.VMEM((EPT_D,), jnp.int32),
            pltpu.VMEM((DCH,), jnp.float32),
            pltpu.VMEM((DCH,), jnp.float32),
            pltpu.SemaphoreType.DMA,
            pltpu.SemaphoreType.DMA,
        ],
    )
    return kern(dst, ext)


def _combine_body(p_ref, d_ref, t_ref, o_ref):
    eps = 1e-16
    wv = p_ref[0] + p_ref[1]
    den = jnp.sum(d_ref[...], axis=0)
    rep = jnp.dot(1.0 / (den + eps), t_ref[...],
                  preferred_element_type=jnp.float32)
    o_ref[...] = wv * rep


def _combine(pm, denp, t_ind):
    bn = 1000
    return pl.pallas_call(
        _combine_body,
        grid=(N_NODES // bn,),
        in_specs=[
            pl.BlockSpec((NUM_CORES, bn, HD), lambda i: (0, i, 0)),
            pl.BlockSpec((NUM_TILES, bn, N_HEADS), lambda i: (0, i, 0)),
            pl.BlockSpec((N_HEADS, HD), lambda i: (0, 0)),
        ],
        out_specs=pl.BlockSpec((bn, HD), lambda i: (i, 0)),
        out_shape=jax.ShapeDtypeStruct((N_NODES, HD), jnp.float32),
    )(pm, denp, t_ind)


def kernel(x, edge_attr, edge_index, Wq, bq, Wk, bk, We, be, Wv, bv):
    scale = 1.0 / np.sqrt(D_HEAD)
    w_all = jnp.concatenate([Wq * scale, Wk, Wv], axis=1)
    b_all = jnp.concatenate([bq * scale, bk, bv]).reshape(1, 3 * HD)

    proj = _projections(x, w_all, b_all)
    q_t = proj[:, :HD]
    k_t = proj[:, HD:2 * HD]
    v_t = proj[:, 2 * HD:]

    src = edge_index[0]
    dst = edge_index[1]
    n_extra = E_PAD - E_NUM
    zpad = jnp.zeros((n_extra,), dtype=src.dtype)
    src_pad = jnp.concatenate([src, zpad])
    dstg_pad = jnp.concatenate([dst, zpad])
    dsts_pad = jnp.concatenate([dst, zpad + N_NODES])

    p = _pmul(src_pad, dstg_pad, k_t, q_t)

    s16 = np.zeros((HD, 16), dtype=np.float32)
    for h in range(N_HEADS):
        s16[h * D_HEAD:(h + 1) * D_HEAD, h] = 1.0
    ex = _escore(edge_attr, p, We, be.reshape(1, HD), jnp.asarray(s16))
    ex_pad2d = jnp.concatenate([ex, jnp.zeros((n_extra, 16), jnp.float32)])
    ex_pad = ex_pad2d.reshape(E_PAD * 16)
    ext_pad = _transpose_ex(ex_pad2d).reshape(16 * E_PAD)

    denp = _den(dstg_pad, ext_pad)
    pm = _scatter(src_pad, dsts_pad, ex_pad, v_t)

    t_ind = np.zeros((N_HEADS, HD), dtype=np.float32)
    for h in range(N_HEADS):
        t_ind[h, h * D_HEAD:(h + 1) * D_HEAD] = 1.0
    wv = _combine(pm, denp.reshape(NUM_TILES, N_NODES, N_HEADS),
                  jnp.asarray(t_ind))

    return wv.reshape(N_NODES, N_HEADS, D_HEAD)
